# speedup vs baseline: 28.4318x; 28.4318x over previous
import jax
import jax.numpy as jnp
from jax.experimental import pallas as pl
from jax.experimental.pallas import tpu as pltpu

_SCORE_THR = 0.1
_BLOCK_R = 512


def _amce_block(x_ref, lab_ref, out_ref):
    step = pl.program_id(0)
    x = x_ref[...]
    lab = lab_ref[...]
    m = jnp.max(x, axis=1, keepdims=True)
    thr = jax.nn.sigmoid(m) - _SCORE_THR
    trow = jnp.where(thr > 0.0, jnp.log(thr) - jnp.log1p(-thr), -jnp.inf)
    col = jax.lax.broadcasted_iota(jnp.int32, x.shape, 1)
    is_lab = col == lab
    t = jnp.exp(-jnp.abs(x))
    bce0 = jnp.maximum(x, 0.0) + jnp.log1p(t)
    w = (x > trow) | is_lab
    contrib = jnp.where(w, bce0, 0.0) - jnp.where(is_lab, x, 0.0)
    part = jnp.sum(contrib)

    @pl.when(step == 0)
    def _init():
        out_ref[0, 0] = 0.0

    out_ref[0, 0] += part


def kernel(cls_logits, labels):
    n_i, n_c = cls_logits.shape
    labs = labels.reshape(n_i, 1)
    grid = n_i // _BLOCK_R
    out = pl.pallas_call(
        _amce_block,
        grid=(grid,),
        in_specs=[
            pl.BlockSpec((_BLOCK_R, n_c), lambda i: (i, 0)),
            pl.BlockSpec((_BLOCK_R, 1), lambda i: (i, 0)),
        ],
        out_specs=pl.BlockSpec((1, 1), lambda i: (0, 0),
                               memory_space=pltpu.SMEM),
        out_shape=jax.ShapeDtypeStruct((1, 1), jnp.float32),
    )(cls_logits, labs)
    return out[0, 0] / jnp.float32(n_i)


# speedup vs baseline: 30.4461x; 1.0708x over previous
import jax
import jax.numpy as jnp
from jax.experimental import pallas as pl
from jax.experimental.pallas import tpu as pltpu

_SCORE_THR = 0.1
_BLOCK_R = 512


def _amce_block(x_ref, lab_ref, out_ref):
    step = pl.program_id(0)
    x = x_ref[...]
    lab = lab_ref[...]
    m = jnp.max(x, axis=1, keepdims=True)
    thr = jax.nn.sigmoid(m) - _SCORE_THR
    trow = jnp.where(thr > 0.0, jnp.log(thr) - jnp.log1p(-thr), -jnp.inf)
    col = jax.lax.broadcasted_iota(jnp.int32, x.shape, 1)
    is_lab = col == lab
    a = jnp.abs(x)
    t = jnp.exp2(a * jnp.float32(-1.4426950408889634))
    bce0 = jnp.maximum(x, 0.0) + jnp.float32(0.6931471805599453) * jnp.log2(1.0 + t)
    w = (x > trow) | is_lab
    contrib = jnp.where(w, bce0, 0.0) - jnp.where(is_lab, x, 0.0)
    part = jnp.sum(contrib)

    @pl.when(step == 0)
    def _init():
        out_ref[0, 0] = 0.0

    out_ref[0, 0] += part


def kernel(cls_logits, labels):
    n_i, n_c = cls_logits.shape
    labs = labels.reshape(n_i, 1)
    grid = n_i // _BLOCK_R
    out = pl.pallas_call(
        _amce_block,
        grid=(grid,),
        in_specs=[
            pl.BlockSpec((_BLOCK_R, n_c), lambda i: (i, 0)),
            pl.BlockSpec((_BLOCK_R, 1), lambda i: (i, 0)),
        ],
        out_specs=pl.BlockSpec((1, 1), lambda i: (0, 0),
                               memory_space=pltpu.SMEM),
        out_shape=jax.ShapeDtypeStruct((1, 1), jnp.float32),
    )(cls_logits, labs)
    return out[0, 0] / jnp.float32(n_i)
